# Initial kernel scaffold; baseline (speedup 1.0000x reference)
import jax
import jax.numpy as jnp
from jax.experimental import pallas as pl


def kernel(x, token_emb, pos_emb):
    raise NotImplementedError("write your pallas kernel here")


# speedup vs baseline: 1.4477x; 1.4477x over previous
import functools

import jax
import jax.numpy as jnp
from jax import lax
from jax.experimental import pallas as pl
from jax.experimental.pallas import tpu as pltpu
from jax.experimental.pallas import tpu_sc as plsc

B = 4096
L = 200
E = 32
N = B * L
NW = 32
PER_W = N // NW
G = 128
GPC = 25
CHUNK = G * GPC
NCHUNK = PER_W // CHUNK
REPS = CHUNK // L

_mesh = plsc.VectorSubcoreMesh(core_axis_name="c", subcore_axis_name="s")


@functools.partial(
    pl.kernel,
    mesh=_mesh,
    out_type=jax.ShapeDtypeStruct((N, E), jnp.float32),
    scratch_types=[
        pltpu.VMEM((CHUNK,), jnp.int32),
        pltpu.VMEM((CHUNK, E), jnp.float32),
        pltpu.VMEM((L, E), jnp.float32),
        pltpu.SemaphoreType.DMA,
    ],
    compiler_params=pltpu.CompilerParams(use_tc_tiling_on_sc=False),
)
def _tok_pos_embed(x1d, tok, pos, out, idx_v, buf, pos_v, sem):
    wid = lax.axis_index("s") * 2 + lax.axis_index("c")
    pltpu.sync_copy(pos, pos_v)

    def chunk_body(c, carry):
        base = wid * PER_W + c * CHUNK
        pltpu.sync_copy(x1d.at[pl.ds(base, CHUNK)], idx_v)
        cps = [
            pltpu.async_copy(
                tok.at[idx_v.at[pl.ds(j * G, G)]],
                buf.at[pl.ds(j * G, G)],
                sem,
            )
            for j in range(GPC)
        ]
        for cp in cps:
            cp.wait()

        def add_body(l, inner):
            p0 = pos_v[l, pl.ds(0, 16)]
            p1 = pos_v[l, pl.ds(16, 16)]
            for k in range(REPS):
                r = l + L * k
                buf[r, pl.ds(0, 16)] += p0
                buf[r, pl.ds(16, 16)] += p1
            return inner

        lax.fori_loop(0, L, add_body, 0)
        pltpu.sync_copy(buf, out.at[pl.ds(base, CHUNK)])
        return carry

    lax.fori_loop(0, NCHUNK, chunk_body, 0)


def kernel(x, token_emb, pos_emb):
    x1d = x.reshape(N).astype(jnp.int32)
    out = _tok_pos_embed(x1d, token_emb, pos_emb)
    return out.reshape(B, L, E)
